# baseline (speedup 1.0000x reference)
import functools

import jax
import jax.numpy as jnp
from jax import lax
from jax.experimental import pallas as pl
from jax.experimental.pallas import tpu as pltpu
from jax.experimental.pallas import tpu_sc as plsc

CHUNK = 128
NBUF = 8
LAG = 2
PAD = 128
RBLK = 256


@functools.lru_cache(maxsize=None)
def _make_format(vocab, dim, nc, ns):
    nw = nc * ns
    nbf = vocab // RBLK
    tail = vocab % RBLK
    base_blocks = nbf // nw
    extra = nbf % nw
    egrp = dim // 16
    tpb = RBLK // 8
    ttl = tail // 8 if tail else 0
    mesh = plsc.VectorSubcoreMesh(
        core_axis_name="c", subcore_axis_name="s", num_cores=nc,
        num_subcores=ns)

    @functools.partial(
        pl.kernel,
        out_type=jax.ShapeDtypeStruct((nbf * tpb + ttl, 8, PAD),
                                      jnp.float32),
        mesh=mesh,
        compiler_params=pltpu.CompilerParams(use_tc_tiling_on_sc=True,
                                             needs_layout_passes=False),
        scratch_types=[
            pltpu.VMEM((2, dim, RBLK), jnp.float32),
            pltpu.VMEM((2, tpb, 8, PAD), jnp.float32),
            pltpu.SemaphoreType.DMA((2,)),
            pltpu.SemaphoreType.DMA((2,)),
        ],
    )
    def format_kernel(tab_t, tail4, out_hbm, in_v, out_v, sem_in, sem_out):
        wid = lax.axis_index("s") * nc + lax.axis_index("c")
        nblk = base_blocks + (wid < extra).astype(jnp.int32)
        iota = lax.iota(jnp.int32, 16)
        evecs = [jnp.int32(g * 16) + iota for g in range(egrp)]

        def fire_in(t, buf):
            jb = wid + nw * t
            for i in range(dim // 8):
                pltpu.async_copy(
                    tab_t.at[pl.ds(8 * i, 8), pl.ds(jb * RBLK, RBLK)],
                    in_v.at[buf, pl.ds(8 * i, 8)], sem_in.at[buf])

        @pl.when(nblk > 0)
        def _():
            fire_in(0, 0)

        def shuffle(buf):
            @plsc.parallel_loop(0, RBLK // 8, 1, unroll=8)
            def rr_body(rr0):
                for k in range(8):
                    lvec = jnp.full((16,), rr0 * 8 + k, jnp.int32)
                    for g in range(egrp):
                        vals = plsc.load_gather(
                            in_v.at[buf], [evecs[g], lvec])
                        out_v[buf, rr0, k, pl.ds(16 * g, 16)] = vals

        def blk(t, carry):
            buf = lax.rem(t, 2)

            @pl.when(t + 1 < nblk)
            def _():
                fire_in(t + 1, 1 - buf)

            for i in range(dim // 8):
                pltpu.make_async_copy(
                    tab_t.at[pl.ds(8 * i, 8), pl.ds(0, RBLK)],
                    in_v.at[buf, pl.ds(8 * i, 8)], sem_in.at[buf]).wait()

            @pl.when(t >= 2)
            def _():
                pltpu.make_async_copy(
                    out_v.at[buf], out_hbm.at[pl.ds(0, tpb)],
                    sem_out.at[buf]).wait()

            shuffle(buf)
            jb = wid + nw * t
            pltpu.async_copy(out_v.at[buf],
                             out_hbm.at[pl.ds(jb * tpb, tpb)],
                             sem_out.at[buf])
            return carry

        lax.fori_loop(0, nblk, blk, 0)

        for b in range(2):
            @pl.when(nblk > b)
            def _():
                pltpu.make_async_copy(
                    out_v.at[b], out_hbm.at[pl.ds(0, tpb)],
                    sem_out.at[b]).wait()

        if tail:
            @pl.when(wid == 0)
            def _():
                pltpu.sync_copy(tail4, out_hbm.at[pl.ds(nbf * tpb, ttl)])

    return format_kernel


@functools.lru_cache(maxsize=None)
def _make_gather(vocab2, dim, cpw, nc, ns):
    nw = nc * ns
    mesh = plsc.VectorSubcoreMesh(
        core_axis_name="c", subcore_axis_name="s", num_cores=nc,
        num_subcores=ns)

    @functools.partial(
        pl.kernel,
        out_type=jax.ShapeDtypeStruct((nw * cpw * CHUNK, PAD), jnp.float32),
        mesh=mesh,
        compiler_params=pltpu.CompilerParams(use_tc_tiling_on_sc=False),
        scratch_types=[
            pltpu.VMEM((cpw, CHUNK), jnp.int32),
            pltpu.VMEM((NBUF, CHUNK, dim), jnp.float32),
            pltpu.SemaphoreType.DMA((NBUF,)),
            pltpu.SemaphoreType.DMA((NBUF,)),
        ],
    )
    def gather_kernel(idx_hbm, table_hbm, out_hbm, idx_v, rows_v, sem_g,
                      sem_o):
        wid = lax.axis_index("s") * nc + lax.axis_index("c")
        row0 = wid * cpw
        pltpu.sync_copy(idx_hbm.at[pl.ds(row0, cpw)], idx_v)

        for b in range(NBUF):
            pltpu.async_copy(table_hbm.at[idx_v.at[b]], rows_v.at[b],
                             sem_g.at[b])

        def step(j, carry):
            jmod = lax.rem(j, NBUF)
            out_slice = out_hbm.at[pl.ds((row0 + j) * CHUNK, CHUNK),
                                   pl.ds(0, dim)]
            pltpu.make_async_copy(
                table_hbm.at[idx_v.at[j]], rows_v.at[jmod],
                sem_g.at[jmod]).wait()
            pltpu.async_copy(rows_v.at[jmod], out_slice, sem_o.at[jmod])

            @pl.when(jnp.logical_and(j >= LAG, j + NBUF - LAG < cpw))
            def _():
                b2 = lax.rem(j - LAG, NBUF)
                pltpu.make_async_copy(rows_v.at[b2], out_slice,
                                      sem_o.at[b2]).wait()
                pltpu.async_copy(table_hbm.at[idx_v.at[j + NBUF - LAG]],
                                 rows_v.at[b2], sem_g.at[b2])

            return carry

        lax.fori_loop(0, cpw, step, 0)

        for b in range(NBUF):
            pltpu.make_async_copy(
                rows_v.at[b],
                out_hbm.at[pl.ds(row0 * CHUNK, CHUNK), pl.ds(0, dim)],
                sem_o.at[b]).wait()

    return gather_kernel


def kernel(x, table):
    vocab, dim = table.shape
    orig_shape = x.shape
    flat = x.reshape(-1).astype(jnp.int32)
    b = flat.shape[0]
    info = plsc.get_sparse_core_info()
    nc, ns = info.num_cores, info.num_subcores
    nw = nc * ns
    per_call = nw * CHUNK
    b_pad = ((b + per_call - 1) // per_call) * per_call
    if b_pad != b:
        flat = jnp.pad(flat, (0, b_pad - b))
    cpw = b_pad // per_call
    idx2d = (flat * (PAD // dim)).reshape(cpw * nw, CHUNK)

    nbf = vocab // RBLK
    tail = vocab % RBLK
    if tail:
        tail4 = jnp.pad(table[nbf * RBLK:],
                        ((0, 0), (0, PAD - dim))).reshape(tail // 8, 8, PAD)
    else:
        tail4 = jnp.zeros((1, 8, PAD), jnp.float32)
    slab = _make_format(vocab, dim, nc, ns)(jnp.transpose(table), tail4)
    table2 = slab.reshape(slab.shape[0] * 8 * (PAD // dim), dim)

    out128 = _make_gather(table2.shape[0], dim, cpw, nc, ns)(idx2d, table2)
    out = out128[:b, :dim]
    return out.reshape(orig_shape + (dim,))


# speedup vs baseline: 1.0066x; 1.0066x over previous
import functools

import jax
import jax.numpy as jnp
from jax import lax
from jax.experimental import pallas as pl
from jax.experimental.pallas import tpu as pltpu
from jax.experimental.pallas import tpu_sc as plsc

CHUNK = 128
NBUF = 8
LAG = 2
PAD = 128
RBLK = 256


@functools.lru_cache(maxsize=None)
def _make_format(vocab, dim, nc, ns):
    nw = nc * ns
    nbf = vocab // RBLK
    tail = vocab % RBLK
    base_blocks = nbf // nw
    extra = nbf % nw
    egrp = dim // 16
    tpb = RBLK // 8
    ttl = tail // 8 if tail else 0
    mesh = plsc.VectorSubcoreMesh(
        core_axis_name="c", subcore_axis_name="s", num_cores=nc,
        num_subcores=ns)

    @functools.partial(
        pl.kernel,
        out_type=jax.ShapeDtypeStruct((nbf * tpb + ttl, 8, PAD),
                                      jnp.float32),
        mesh=mesh,
        compiler_params=pltpu.CompilerParams(use_tc_tiling_on_sc=True,
                                             needs_layout_passes=False),
        scratch_types=[
            pltpu.VMEM((2, dim, RBLK), jnp.float32),
            pltpu.VMEM((2, tpb, 8, PAD), jnp.float32),
            pltpu.SemaphoreType.DMA((2,)),
            pltpu.SemaphoreType.DMA((2,)),
        ],
    )
    def format_kernel(tab_t, tail4, out_hbm, in_v, out_v, sem_in, sem_out):
        wid = lax.axis_index("s") * nc + lax.axis_index("c")
        nblk = base_blocks + (wid < extra).astype(jnp.int32)
        iota = lax.iota(jnp.int32, 16)
        evecs = [jnp.int32(g * 16) + iota for g in range(egrp)]

        def fire_in(t, buf):
            jb = wid + nw * t
            for i in range(dim // 8):
                pltpu.async_copy(
                    tab_t.at[pl.ds(8 * i, 8), pl.ds(jb * RBLK, RBLK)],
                    in_v.at[buf, pl.ds(8 * i, 8)], sem_in.at[buf])

        @pl.when(nblk > 0)
        def _():
            fire_in(0, 0)

        def shuffle(buf):
            @plsc.parallel_loop(0, RBLK // 8, 1, unroll=4)
            def rr_body(rr0):
                for k in range(8):
                    lvec = jnp.full((16,), rr0 * 8 + k, jnp.int32)
                    for g in range(egrp):
                        vals = plsc.load_gather(
                            in_v.at[buf], [evecs[g], lvec])
                        out_v[buf, rr0, k, pl.ds(16 * g, 16)] = vals

        def blk(t, carry):
            buf = lax.rem(t, 2)

            @pl.when(t + 1 < nblk)
            def _():
                fire_in(t + 1, 1 - buf)

            for i in range(dim // 8):
                pltpu.make_async_copy(
                    tab_t.at[pl.ds(8 * i, 8), pl.ds(0, RBLK)],
                    in_v.at[buf, pl.ds(8 * i, 8)], sem_in.at[buf]).wait()

            @pl.when(t >= 2)
            def _():
                pltpu.make_async_copy(
                    out_v.at[buf], out_hbm.at[pl.ds(0, tpb)],
                    sem_out.at[buf]).wait()

            shuffle(buf)
            jb = wid + nw * t
            pltpu.async_copy(out_v.at[buf],
                             out_hbm.at[pl.ds(jb * tpb, tpb)],
                             sem_out.at[buf])
            return carry

        lax.fori_loop(0, nblk, blk, 0)

        for b in range(2):
            @pl.when(nblk > b)
            def _():
                pltpu.make_async_copy(
                    out_v.at[b], out_hbm.at[pl.ds(0, tpb)],
                    sem_out.at[b]).wait()

        if tail:
            @pl.when(wid == 0)
            def _():
                pltpu.sync_copy(tail4, out_hbm.at[pl.ds(nbf * tpb, ttl)])

    return format_kernel


@functools.lru_cache(maxsize=None)
def _make_gather(vocab2, dim, cpw, nc, ns):
    nw = nc * ns
    mesh = plsc.VectorSubcoreMesh(
        core_axis_name="c", subcore_axis_name="s", num_cores=nc,
        num_subcores=ns)

    @functools.partial(
        pl.kernel,
        out_type=jax.ShapeDtypeStruct((nw * cpw * CHUNK, PAD), jnp.float32),
        mesh=mesh,
        compiler_params=pltpu.CompilerParams(use_tc_tiling_on_sc=False),
        scratch_types=[
            pltpu.VMEM((cpw, CHUNK), jnp.int32),
            pltpu.VMEM((NBUF, CHUNK, dim), jnp.float32),
            pltpu.SemaphoreType.DMA((NBUF,)),
            pltpu.SemaphoreType.DMA((NBUF,)),
        ],
    )
    def gather_kernel(idx_hbm, table_hbm, out_hbm, idx_v, rows_v, sem_g,
                      sem_o):
        wid = lax.axis_index("s") * nc + lax.axis_index("c")
        row0 = wid * cpw
        pltpu.sync_copy(idx_hbm.at[pl.ds(row0, cpw)], idx_v)

        for b in range(NBUF):
            pltpu.async_copy(table_hbm.at[idx_v.at[b]], rows_v.at[b],
                             sem_g.at[b])

        def step(j, carry):
            jmod = lax.rem(j, NBUF)
            out_slice = out_hbm.at[pl.ds((row0 + j) * CHUNK, CHUNK),
                                   pl.ds(0, dim)]
            pltpu.make_async_copy(
                table_hbm.at[idx_v.at[j]], rows_v.at[jmod],
                sem_g.at[jmod]).wait()
            pltpu.async_copy(rows_v.at[jmod], out_slice, sem_o.at[jmod])

            @pl.when(jnp.logical_and(j >= LAG, j + NBUF - LAG < cpw))
            def _():
                b2 = lax.rem(j - LAG, NBUF)
                pltpu.make_async_copy(rows_v.at[b2], out_slice,
                                      sem_o.at[b2]).wait()
                pltpu.async_copy(table_hbm.at[idx_v.at[j + NBUF - LAG]],
                                 rows_v.at[b2], sem_g.at[b2])

            return carry

        lax.fori_loop(0, cpw, step, 0)

        for b in range(NBUF):
            pltpu.make_async_copy(
                rows_v.at[b],
                out_hbm.at[pl.ds(row0 * CHUNK, CHUNK), pl.ds(0, dim)],
                sem_o.at[b]).wait()

    return gather_kernel


def kernel(x, table):
    vocab, dim = table.shape
    orig_shape = x.shape
    flat = x.reshape(-1).astype(jnp.int32)
    b = flat.shape[0]
    info = plsc.get_sparse_core_info()
    nc, ns = info.num_cores, info.num_subcores
    nw = nc * ns
    per_call = nw * CHUNK
    b_pad = ((b + per_call - 1) // per_call) * per_call
    if b_pad != b:
        flat = jnp.pad(flat, (0, b_pad - b))
    cpw = b_pad // per_call
    idx2d = (flat * (PAD // dim)).reshape(cpw * nw, CHUNK)

    nbf = vocab // RBLK
    tail = vocab % RBLK
    if tail:
        tail4 = jnp.pad(table[nbf * RBLK:],
                        ((0, 0), (0, PAD - dim))).reshape(tail // 8, 8, PAD)
    else:
        tail4 = jnp.zeros((1, 8, PAD), jnp.float32)
    slab = _make_format(vocab, dim, nc, ns)(jnp.transpose(table), tail4)
    table2 = slab.reshape(slab.shape[0] * 8 * (PAD // dim), dim)

    out128 = _make_gather(table2.shape[0], dim, cpw, nc, ns)(idx2d, table2)
    out = out128[:b, :dim]
    return out.reshape(orig_shape + (dim,))


# speedup vs baseline: 1.3702x; 1.3612x over previous
import functools

import jax
import jax.numpy as jnp
from jax import lax
from jax.experimental import pallas as pl
from jax.experimental.pallas import tpu as pltpu
from jax.experimental.pallas import tpu_sc as plsc

CHUNK = 128
NBUF = 8
LAG = 2
PAD = 128


@functools.lru_cache(maxsize=None)
def _make_gather(vocab2, dim, cpw, nc, ns):
    nw = nc * ns
    mesh = plsc.VectorSubcoreMesh(
        core_axis_name="c", subcore_axis_name="s", num_cores=nc,
        num_subcores=ns)

    @functools.partial(
        pl.kernel,
        out_type=jax.ShapeDtypeStruct((nw * cpw * CHUNK, PAD), jnp.float32),
        mesh=mesh,
        compiler_params=pltpu.CompilerParams(use_tc_tiling_on_sc=False),
        scratch_types=[
            pltpu.VMEM((cpw, CHUNK), jnp.int32),
            pltpu.VMEM((NBUF, CHUNK, dim), jnp.float32),
            pltpu.SemaphoreType.DMA((NBUF,)),
            pltpu.SemaphoreType.DMA((NBUF,)),
        ],
    )
    def gather_kernel(idx_hbm, table_hbm, out_hbm, idx_v, rows_v, sem_g,
                      sem_o):
        wid = lax.axis_index("s") * nc + lax.axis_index("c")
        row0 = wid * cpw
        pltpu.sync_copy(idx_hbm.at[pl.ds(row0, cpw)], idx_v)

        for b in range(NBUF):
            pltpu.async_copy(table_hbm.at[idx_v.at[b]], rows_v.at[b],
                             sem_g.at[b])

        def step(j, carry):
            jmod = lax.rem(j, NBUF)
            out_slice = out_hbm.at[pl.ds((row0 + j) * CHUNK, CHUNK),
                                   pl.ds(0, dim)]
            pltpu.make_async_copy(
                table_hbm.at[idx_v.at[j]], rows_v.at[jmod],
                sem_g.at[jmod]).wait()
            pltpu.async_copy(rows_v.at[jmod], out_slice, sem_o.at[jmod])

            @pl.when(jnp.logical_and(j >= LAG, j + NBUF - LAG < cpw))
            def _():
                b2 = lax.rem(j - LAG, NBUF)
                pltpu.make_async_copy(rows_v.at[b2], out_slice,
                                      sem_o.at[b2]).wait()
                pltpu.async_copy(table_hbm.at[idx_v.at[j + NBUF - LAG]],
                                 rows_v.at[b2], sem_g.at[b2])

            return carry

        lax.fori_loop(0, cpw, step, 0)

        for b in range(NBUF):
            pltpu.make_async_copy(
                rows_v.at[b],
                out_hbm.at[pl.ds(row0 * CHUNK, CHUNK), pl.ds(0, dim)],
                sem_o.at[b]).wait()

    return gather_kernel


def kernel(x, table):
    vocab, dim = table.shape
    orig_shape = x.shape
    flat = x.reshape(-1).astype(jnp.int32)
    b = flat.shape[0]
    info = plsc.get_sparse_core_info()
    nc, ns = info.num_cores, info.num_subcores
    nw = nc * ns
    per_call = nw * CHUNK
    b_pad = ((b + per_call - 1) // per_call) * per_call
    if b_pad != b:
        flat = jnp.pad(flat, (0, b_pad - b))
    cpw = b_pad // per_call
    idx2d = (flat * (PAD // dim)).reshape(cpw * nw, CHUNK)
    table_pad = jnp.concatenate(
        [table, jnp.zeros((vocab, PAD - dim), jnp.float32)], axis=1)
    table2 = table_pad.reshape(vocab * (PAD // dim), dim)
    out128 = _make_gather(table2.shape[0], dim, cpw, nc, ns)(idx2d, table2)
    out = out128.reshape(b_pad // CHUNK * CHUNK, PAD)[:b, :dim]
    return out.reshape(orig_shape + (dim,))


# speedup vs baseline: 1.3760x; 1.0043x over previous
import functools

import jax
import jax.numpy as jnp
from jax import lax
from jax.experimental import pallas as pl
from jax.experimental.pallas import tpu as pltpu
from jax.experimental.pallas import tpu_sc as plsc

CHUNK = 128
NBUF = 8
LAG = 2
PAD = 128


@functools.lru_cache(maxsize=None)
def _make_gather(vocab2, dim, cpw, nc, ns):
    nw = nc * ns
    mesh = plsc.VectorSubcoreMesh(
        core_axis_name="c", subcore_axis_name="s", num_cores=nc,
        num_subcores=ns)

    @functools.partial(
        pl.kernel,
        out_type=jax.ShapeDtypeStruct((nw * cpw * CHUNK, PAD), jnp.float32),
        mesh=mesh,
        compiler_params=pltpu.CompilerParams(use_tc_tiling_on_sc=False),
        scratch_types=[
            pltpu.VMEM((cpw, CHUNK), jnp.int32),
            pltpu.VMEM((NBUF, CHUNK, dim), jnp.float32),
            pltpu.SemaphoreType.DMA((NBUF,)),
            pltpu.SemaphoreType.DMA((NBUF,)),
        ],
    )
    def gather_kernel(idx_hbm, table_hbm, out_hbm, idx_v, rows_v, sem_g,
                      sem_o):
        wid = lax.axis_index("s") * nc + lax.axis_index("c")
        row0 = wid * cpw
        pltpu.sync_copy(idx_hbm.at[pl.ds(row0, cpw)], idx_v)

        for b in range(NBUF):
            pltpu.async_copy(table_hbm.at[idx_v.at[b]], rows_v.at[b],
                             sem_g.at[b])

        def step(j, carry):
            jmod = lax.rem(j, NBUF)
            out_slice = out_hbm.at[pl.ds((row0 + j) * CHUNK, CHUNK),
                                   pl.ds(0, dim)]
            pltpu.make_async_copy(
                table_hbm.at[idx_v.at[j]], rows_v.at[jmod],
                sem_g.at[jmod]).wait()
            pltpu.async_copy(rows_v.at[jmod], out_slice, sem_o.at[jmod])

            @pl.when(jnp.logical_and(j >= LAG, j + NBUF - LAG < cpw))
            def _():
                b2 = lax.rem(j - LAG, NBUF)
                pltpu.make_async_copy(rows_v.at[b2], out_slice,
                                      sem_o.at[b2]).wait()
                pltpu.async_copy(table_hbm.at[idx_v.at[j + NBUF - LAG]],
                                 rows_v.at[b2], sem_g.at[b2])

            return carry

        lax.fori_loop(0, cpw, step, 0)

        for b in range(NBUF):
            pltpu.make_async_copy(
                rows_v.at[b],
                out_hbm.at[pl.ds(row0 * CHUNK, CHUNK), pl.ds(0, dim)],
                sem_o.at[b]).wait()

    return gather_kernel


def kernel(x, table):
    vocab, dim = table.shape
    orig_shape = x.shape
    flat = x.reshape(-1).astype(jnp.int32)
    b = flat.shape[0]
    info = plsc.get_sparse_core_info()
    nc, ns = info.num_cores, info.num_subcores
    nw = nc * ns
    per_call = nw * CHUNK
    b_pad = ((b + per_call - 1) // per_call) * per_call
    if b_pad != b:
        flat = jnp.pad(flat, (0, b_pad - b))
    cpw = b_pad // per_call
    idx2d = (flat * (PAD // dim)).reshape(cpw * nw, CHUNK)
    table_pad = jnp.pad(table, ((0, 0), (0, PAD - dim)))
    table2 = table_pad.reshape(vocab * (PAD // dim), dim)
    out128 = _make_gather(table2.shape[0], dim, cpw, nc, ns)(idx2d, table2)
    out = out128.reshape(b_pad // CHUNK * CHUNK, PAD)[:b, :dim]
    return out.reshape(orig_shape + (dim,))
